# baseline (speedup 1.0000x reference)
import functools

import jax
import jax.numpy as jnp
from jax import lax
from jax.experimental import pallas as pl
from jax.experimental.pallas import tpu as pltpu
from jax.experimental.pallas import tpu_sc as plsc

_NUM_DOMAINS = 26
_VOCAB = 100000
_VOCAB_PAD = 100096
_EMB = 16
_BATCH = 16384

_NUM_CORES = 2
_NUM_SUBCORES = 16
_NW = _NUM_CORES * _NUM_SUBCORES
_DPS = _NUM_DOMAINS // _NUM_CORES

_SHARD = 6272
_Q = 896
_CHUNKS = (_Q,) * 7
_TAIL_CHUNKS = (_Q,) * 6 + (512,)
_TAIL32_BASE = 99968

_ROWS_W = _BATCH // _NW
_CHUNK = 128
_NCHUNK = _ROWS_W // _CHUNK
_UNROLL = 4
_LANES = 16

_sc_mesh = plsc.VectorSubcoreMesh(
    core_axis_name="c", subcore_axis_name="s",
    num_cores=_NUM_CORES, num_subcores=_NUM_SUBCORES)


@functools.partial(
    pl.kernel,
    out_type=jax.ShapeDtypeStruct((_NUM_DOMAINS, _VOCAB_PAD // 8, 128), jnp.float32),
    mesh=_sc_mesh,
    scratch_types=[
        pltpu.VMEM((2, _EMB, _Q), jnp.float32),
        pltpu.VMEM((2, _Q // 8, 128), jnp.float32),
        pltpu.VMEM((_EMB, 32), jnp.float32),
        pltpu.SemaphoreType.DMA,
        pltpu.SemaphoreType.DMA,
        pltpu.SemaphoreType.DMA,
    ],
    compiler_params=pltpu.CompilerParams(
        use_tc_tiling_on_sc=True, needs_layout_passes=False),
)
def _sc_relayout(t2_hbm, t2tail_hbm, scratch_hbm, tbuf, linbuf, ttail,
                 sem_in, sem_o0, sem_o1):
    cid = lax.axis_index("c")
    sid = lax.axis_index("s")
    iota = lax.iota(jnp.int32, _LANES)
    vbase = pl.multiple_of(sid * _SHARD, 128)

    def do_chunk(i_dyn, off, width, buf_slot):
        src = t2_hbm.at[i_dyn, :, pl.ds(off, width)]
        pltpu.async_copy(src, tbuf.at[buf_slot, :, pl.ds(0, width)], sem_in).wait()

        def body(v0, _):
            for u in range(_LANES):
                v = v0 * _LANES + u
                row = plsc.load_gather(
                    tbuf.at[buf_slot],
                    [iota, jnp.full((_LANES,), v, jnp.int32)])
                linbuf[buf_slot, (v0 * _LANES + u) // 8, pl.ds((v % 8) * _EMB, _EMB)] = row
            return 0

        lax.fori_loop(0, width // _LANES, body, 0)
        r0 = (off >> 3) if isinstance(off, int) else pl.multiple_of(off >> 3, 8)
        dst = scratch_hbm.at[i_dyn, pl.ds(r0, width // 8), :]
        sem = sem_o0 if buf_slot == 0 else sem_o1
        return pltpu.async_copy(linbuf.at[buf_slot, pl.ds(0, width // 8), :], dst, sem)

    def do_tail32(i_dyn, buf_slot):
        pltpu.async_copy(t2tail_hbm.at[i_dyn], ttail, sem_in).wait()

        def body(v0, _):
            for u in range(_LANES):
                v = v0 * _LANES + u
                row = plsc.load_gather(
                    ttail, [iota, jnp.full((_LANES,), v, jnp.int32)])
                linbuf[buf_slot, v // 8, pl.ds((v % 8) * _EMB, _EMB)] = row
            return 0

        lax.fori_loop(0, 32 // _LANES, body, 0)
        dst = scratch_hbm.at[i_dyn, pl.ds(_TAIL32_BASE // 8, 8), :]
        sem = sem_o0 if buf_slot == 0 else sem_o1
        return pltpu.async_copy(linbuf.at[buf_slot, pl.ds(0, 8), :], dst, sem)

    def run_domain(i_dyn, widths, off, tail32):
        waits = []
        for k, w in enumerate(widths):
            if k >= 2:
                waits[k - 2].wait()
            waits.append(do_chunk(i_dyn, off, w, k % 2))
            off += w
        if tail32:
            k = len(widths)
            waits[k - 2].wait()
            waits.append(do_tail32(i_dyn, k % 2))
        waits[-2].wait()
        waits[-1].wait()

    def per_domain(j, _):
        i_dyn = cid * _DPS + j

        @pl.when(sid < 15)
        def _():
            run_domain(i_dyn, _CHUNKS, vbase, False)

        @pl.when(sid == 15)
        def _():
            run_domain(i_dyn, _TAIL_CHUNKS, 15 * _SHARD, True)

        return 0

    lax.fori_loop(0, _DPS, per_domain, 0)


@functools.partial(
    pl.kernel,
    out_type=jax.ShapeDtypeStruct((_BATCH, _EMB), jnp.float32),
    mesh=_sc_mesh,
    scratch_types=[
        pltpu.VMEM((_ROWS_W, _NUM_DOMAINS), jnp.int32),
        pltpu.VMEM((_NUM_DOMAINS, _ROWS_W), jnp.int32),
        pltpu.VMEM((_NUM_DOMAINS, _CHUNK, _EMB), jnp.float32),
        pltpu.VMEM((_CHUNK, _EMB), jnp.float32),
        pltpu.SemaphoreType.DMA,
    ],
    compiler_params=pltpu.CompilerParams(
        use_tc_tiling_on_sc=False, needs_layout_passes=False),
)
def _sc_gather_sum(x_hbm, *rest):
    tables = rest[:_NUM_DOMAINS]
    out_hbm, xw, idx_v, buf, acc, sem = rest[_NUM_DOMAINS:]
    wid = lax.axis_index("s") * _NUM_CORES + lax.axis_index("c")
    base = wid * _ROWS_W
    pltpu.sync_copy(x_hbm.at[pl.ds(base, _ROWS_W)], xw)

    lane = lax.iota(jnp.int32, _LANES)

    def tbody(v, _):
        r0 = v * _LANES
        rows = r0 + lane
        for i in range(_NUM_DOMAINS):
            col = jnp.full((_LANES,), i, dtype=jnp.int32)
            g = plsc.load_gather(xw, [rows, col])
            idx_v[i, pl.ds(r0, _LANES)] = g
        return 0

    lax.fori_loop(0, _ROWS_W // _LANES, tbody, 0)

    for c in range(_NCHUNK):
        copies = []
        for i in range(_NUM_DOMAINS):
            copies.append(
                pltpu.async_copy(
                    tables[i].at[idx_v.at[i, pl.ds(c * _CHUNK, _CHUNK)]],
                    buf.at[i], sem))
        for cp in copies:
            cp.wait()

        def body(r0, _):
            for u in range(_UNROLL):
                r = r0 * _UNROLL + u
                s = buf[0, r, :]
                for i in range(1, _NUM_DOMAINS):
                    s = s + buf[i, r, :]
                acc[r, :] = s
            return 0

        lax.fori_loop(0, _CHUNK // _UNROLL, body, 0)
        pltpu.sync_copy(acc, out_hbm.at[pl.ds(base + c * _CHUNK, _CHUNK)])


def _finish_body(s_ref, o_ref):
    s = s_ref[...]
    s2 = jnp.sum(s * s, axis=1)
    o_ref[...] = jnp.tanh(jnp.sqrt(s2))


def _finish(summed):
    return pl.pallas_call(
        _finish_body,
        out_shape=jax.ShapeDtypeStruct((_BATCH,), jnp.float32),
    )(summed)


def kernel(x, tables):
    t2 = jnp.transpose(tables, (0, 2, 1))
    t2tail = t2[:, :, _TAIL32_BASE:]
    scratch = _sc_relayout(t2, t2tail)
    per_domain = [
        scratch[i].reshape(_VOCAB_PAD, _EMB)
        for i in range(_NUM_DOMAINS)
    ]
    summed = _sc_gather_sum(x, *per_domain)
    return _finish(summed)


# speedup vs baseline: 1.0798x; 1.0798x over previous
import functools

import jax
import jax.numpy as jnp
from jax import lax
from jax.experimental import pallas as pl
from jax.experimental.pallas import tpu as pltpu
from jax.experimental.pallas import tpu_sc as plsc

_NUM_DOMAINS = 26
_VOCAB = 100000
_VOCAB_PAD = 100096
_EMB = 16
_BATCH = 16384

_NUM_CORES = 2
_NUM_SUBCORES = 16
_NW = _NUM_CORES * _NUM_SUBCORES
_DPS = _NUM_DOMAINS // _NUM_CORES

_SHARD = 6272
_Q = 896
_CHUNKS = (_Q,) * 7
_TAIL_CHUNKS = (_Q,) * 6 + (512,)
_TAIL32_BASE = 99968

_ROWS_W = _BATCH // _NW
_CHUNK = 128
_NCHUNK = _ROWS_W // _CHUNK
_UNROLL = 4
_LANES = 16

_sc_mesh = plsc.VectorSubcoreMesh(
    core_axis_name="c", subcore_axis_name="s",
    num_cores=_NUM_CORES, num_subcores=_NUM_SUBCORES)


@functools.partial(
    pl.kernel,
    out_type=jax.ShapeDtypeStruct((_NUM_DOMAINS, _VOCAB_PAD // 8, 128), jnp.float32),
    mesh=_sc_mesh,
    scratch_types=[
        pltpu.VMEM((2, _Q // 128 * 2, 8, 128), jnp.float32),
        pltpu.VMEM((2, _Q // 8, 128), jnp.float32),
        pltpu.VMEM((_EMB, 32), jnp.float32),
        pltpu.SemaphoreType.DMA,
        pltpu.SemaphoreType.DMA,
        pltpu.SemaphoreType.DMA,
        pltpu.SemaphoreType.DMA,
    ],
    compiler_params=pltpu.CompilerParams(
        use_tc_tiling_on_sc=True, needs_layout_passes=False),
)
def _sc_relayout(t2_hbm, t2tail_hbm, scratch_hbm, tbuf, linbuf, ttail,
                 sem_i0, sem_i1, sem_o0, sem_o1):
    cid = lax.axis_index("c")
    sid = lax.axis_index("s")
    iota = lax.iota(jnp.int32, _LANES)
    vbase = pl.multiple_of(sid * _SHARD, 128)
    tile_of_e = (iota >> 3)
    sub_of_e = iota & 7

    def stage_chunk(i_dyn, off, width, buf_slot):
        ntc = width // 128
        sem = sem_i0 if buf_slot == 0 else sem_i1
        handles = []
        for tr in range(2):
            for tc in range(ntc):
                o = off + tc * 128
                o = o if isinstance(o, int) else pl.multiple_of(o, 128)
                src = t2_hbm.at[i_dyn, pl.ds(tr * 8, 8), pl.ds(o, 128)]
                handles.append(
                    pltpu.async_copy(src, tbuf.at[buf_slot, tr * ntc + tc], sem))
        return handles

    def transpose_chunk(i_dyn, off, width, buf_slot):
        ntc = width // 128

        def body(v0, _):
            for u in range(_LANES):
                v = v0 * _LANES + u
                i1 = tile_of_e * ntc + (v >> 7)
                i3 = jnp.full((_LANES,), v & 127, jnp.int32)
                row = plsc.load_gather(tbuf.at[buf_slot], [i1, sub_of_e, i3])
                linbuf[buf_slot, v // 8, pl.ds((v % 8) * _EMB, _EMB)] = row
            return 0

        lax.fori_loop(0, width // _LANES, body, 0)
        r0 = (off >> 3) if isinstance(off, int) else pl.multiple_of(off >> 3, 8)
        dst = scratch_hbm.at[i_dyn, pl.ds(r0, width // 8), :]
        sem = sem_o0 if buf_slot == 0 else sem_o1
        return pltpu.async_copy(linbuf.at[buf_slot, pl.ds(0, width // 8), :], dst, sem)

    def do_tail32(i_dyn, buf_slot):
        pltpu.async_copy(t2tail_hbm.at[i_dyn], ttail, sem_i0).wait()

        def body(v0, _):
            for u in range(_LANES):
                v = v0 * _LANES + u
                row = plsc.load_gather(
                    ttail, [iota, jnp.full((_LANES,), v, jnp.int32)])
                linbuf[buf_slot, v // 8, pl.ds((v % 8) * _EMB, _EMB)] = row
            return 0

        lax.fori_loop(0, 32 // _LANES, body, 0)
        dst = scratch_hbm.at[i_dyn, pl.ds(_TAIL32_BASE // 8, 8), :]
        sem = sem_o0 if buf_slot == 0 else sem_o1
        return pltpu.async_copy(linbuf.at[buf_slot, pl.ds(0, 8), :], dst, sem)

    def run_domain(i_dyn, widths, offs, tail32):
        in_h = [stage_chunk(i_dyn, offs[0], widths[0], 0)]
        out_h = []
        for k, w in enumerate(widths):
            if k + 1 < len(widths):
                in_h.append(
                    stage_chunk(i_dyn, offs[k + 1], widths[k + 1], (k + 1) % 2))
            for h in in_h[k]:
                h.wait()
            if k >= 2:
                out_h[k - 2].wait()
            out_h.append(transpose_chunk(i_dyn, offs[k], w, k % 2))
        if tail32:
            k = len(widths)
            out_h[k - 2].wait()
            out_h.append(do_tail32(i_dyn, k % 2))
        out_h[-2].wait()
        out_h[-1].wait()

    def per_domain(j, _):
        i_dyn = cid * _DPS + j

        @pl.when(sid < 15)
        def _():
            offs = [vbase + k * _Q for k in range(len(_CHUNKS))]
            run_domain(i_dyn, _CHUNKS, offs, False)

        @pl.when(sid == 15)
        def _():
            base = 15 * _SHARD
            offs, o = [], base
            for w in _TAIL_CHUNKS:
                offs.append(o)
                o += w
            run_domain(i_dyn, _TAIL_CHUNKS, offs, True)

        return 0

    lax.fori_loop(0, _DPS, per_domain, 0)


@functools.partial(
    pl.kernel,
    out_type=jax.ShapeDtypeStruct((_BATCH, _EMB), jnp.float32),
    mesh=_sc_mesh,
    scratch_types=[
        pltpu.VMEM((_ROWS_W, _NUM_DOMAINS), jnp.int32),
        pltpu.VMEM((_NUM_DOMAINS, _ROWS_W), jnp.int32),
        pltpu.VMEM((_NUM_DOMAINS, _CHUNK, _EMB), jnp.float32),
        pltpu.VMEM((_CHUNK, _EMB), jnp.float32),
        pltpu.SemaphoreType.DMA,
    ],
    compiler_params=pltpu.CompilerParams(
        use_tc_tiling_on_sc=False, needs_layout_passes=False),
)
def _sc_gather_sum(x_hbm, *rest):
    tables = rest[:_NUM_DOMAINS]
    out_hbm, xw, idx_v, buf, acc, sem = rest[_NUM_DOMAINS:]
    wid = lax.axis_index("s") * _NUM_CORES + lax.axis_index("c")
    base = wid * _ROWS_W
    pltpu.sync_copy(x_hbm.at[pl.ds(base, _ROWS_W)], xw)

    lane = lax.iota(jnp.int32, _LANES)

    def tbody(v, _):
        r0 = v * _LANES
        rows = r0 + lane
        for i in range(_NUM_DOMAINS):
            col = jnp.full((_LANES,), i, dtype=jnp.int32)
            g = plsc.load_gather(xw, [rows, col])
            idx_v[i, pl.ds(r0, _LANES)] = g
        return 0

    lax.fori_loop(0, _ROWS_W // _LANES, tbody, 0)

    for c in range(_NCHUNK):
        copies = []
        for i in range(_NUM_DOMAINS):
            copies.append(
                pltpu.async_copy(
                    tables[i].at[idx_v.at[i, pl.ds(c * _CHUNK, _CHUNK)]],
                    buf.at[i], sem))
        for cp in copies:
            cp.wait()

        def body(r0, _):
            for u in range(_UNROLL):
                r = r0 * _UNROLL + u
                s = buf[0, r, :]
                for i in range(1, _NUM_DOMAINS):
                    s = s + buf[i, r, :]
                acc[r, :] = s
            return 0

        lax.fori_loop(0, _CHUNK // _UNROLL, body, 0)
        pltpu.sync_copy(acc, out_hbm.at[pl.ds(base + c * _CHUNK, _CHUNK)])


def _finish_body(s_ref, o_ref):
    s = s_ref[...]
    s2 = jnp.sum(s * s, axis=1)
    o_ref[...] = jnp.tanh(jnp.sqrt(s2))


def _finish(summed):
    return pl.pallas_call(
        _finish_body,
        out_shape=jax.ShapeDtypeStruct((_BATCH,), jnp.float32),
    )(summed)


def kernel(x, tables):
    t2 = jnp.transpose(tables, (0, 2, 1))
    t2tail = t2[:, :, _TAIL32_BASE:]
    scratch = _sc_relayout(t2, t2tail)
    per_domain = [
        scratch[i].reshape(_VOCAB_PAD, _EMB)
        for i in range(_NUM_DOMAINS)
    ]
    summed = _sc_gather_sum(x, *per_domain)
    return _finish(summed)


# speedup vs baseline: 1.1894x; 1.1015x over previous
import functools

import jax
import jax.numpy as jnp
from jax import lax
from jax.experimental import pallas as pl
from jax.experimental.pallas import tpu as pltpu
from jax.experimental.pallas import tpu_sc as plsc

_NUM_DOMAINS = 26
_VOCAB = 100000
_EMB = 16
_BATCH = 16384

_NUM_CORES = 2
_NUM_SUBCORES = 16
_NW = _NUM_CORES * _NUM_SUBCORES
_ROWS_W = _BATCH // _NW
_CHUNK = 128
_NCHUNK = _ROWS_W // _CHUNK
_UNROLL = 4


_sc_mesh = plsc.VectorSubcoreMesh(
    core_axis_name="c", subcore_axis_name="s",
    num_cores=_NUM_CORES, num_subcores=_NUM_SUBCORES)


@functools.partial(
    pl.kernel,
    out_type=jax.ShapeDtypeStruct((_BATCH, _EMB), jnp.float32),
    mesh=_sc_mesh,
    scratch_types=[
        pltpu.VMEM((_NUM_DOMAINS, _NCHUNK, _CHUNK), jnp.int32),
        pltpu.VMEM((_NUM_DOMAINS, _CHUNK, _EMB), jnp.float32),
        pltpu.VMEM((_CHUNK, _EMB), jnp.float32),
        pltpu.SemaphoreType.DMA,
    ],
    compiler_params=pltpu.CompilerParams(use_tc_tiling_on_sc=False),
)
def _sc_gather_sum(table_hbm, idx_hbm, out_hbm, idx_v, buf, acc, sem):
    wid = lax.axis_index("s") * _NUM_CORES + lax.axis_index("c")
    base = wid * _ROWS_W
    pltpu.sync_copy(idx_hbm.at[wid], idx_v)
    for c in range(_NCHUNK):
        copies = []
        for i in range(_NUM_DOMAINS):
            copies.append(
                pltpu.async_copy(table_hbm.at[idx_v.at[i, c]], buf.at[i], sem))
        for cp in copies:
            cp.wait()
        def body(r0, _):
            for u in range(_UNROLL):
                r = r0 * _UNROLL + u
                s = buf[0, r, :]
                for i in range(1, _NUM_DOMAINS):
                    s = s + buf[i, r, :]
                acc[r, :] = s
            return 0
        lax.fori_loop(0, _CHUNK // _UNROLL, body, 0)
        pltpu.sync_copy(acc, out_hbm.at[pl.ds(base + c * _CHUNK, _CHUNK)])


def _finish_body(s_ref, o_ref):
    s = s_ref[...]
    s2 = jnp.sum(s * s, axis=1)
    o_ref[...] = jnp.tanh(jnp.sqrt(s2))


def _finish(summed):
    return pl.pallas_call(
        _finish_body,
        out_shape=jax.ShapeDtypeStruct((_BATCH,), jnp.float32),
    )(summed)


def kernel(x, tables):
    flat_table = tables.reshape(_NUM_DOMAINS * _VOCAB, _EMB)
    offs = jnp.arange(_NUM_DOMAINS, dtype=jnp.int32) * _VOCAB
    idx = (x + offs[None, :]).T.reshape(_NUM_DOMAINS, _NW, _ROWS_W)
    idx = idx.transpose(1, 0, 2).reshape(_NW, _NUM_DOMAINS, _NCHUNK, _CHUNK)
    summed = _sc_gather_sum(flat_table, idx)
    return _finish(summed)
